# initial kernel scaffold (unmeasured)
import jax
import jax.numpy as jnp
from jax import lax
from jax.experimental import pallas as pl
from jax.experimental.pallas import tpu as pltpu

CHUNK = 64


def kernel(x, dest):
    t, d = x.shape
    max_chunks = t // CHUNK

    zeros = dest == 0
    c = jnp.cumsum(zeros.astype(jnp.int32))
    c0 = c[-1]
    i = jnp.arange(t, dtype=jnp.int32)
    pos = jnp.where(zeros, c - 1, c0 + i - c)
    order = jnp.zeros((t,), jnp.int32).at[pos].set(i, unique_indices=True)
    s = jnp.take(x, order, axis=0, unique_indices=True)

    c0_arr = jnp.reshape(c0, (1,))

    def body(c0_ref, s_ref, out_ref, send_sems, recv_sems):
        my_x = lax.axis_index("x")
        my_y = lax.axis_index("y")
        my_z = lax.axis_index("z")
        partner = (my_x, 1 - my_y, my_z)

        c0v = c0_ref[0]
        n_send = jnp.where(my_y == 0, t - c0v, c0v)
        send_start = jnp.where(my_y == 0, c0v, 0)
        dst_start = jnp.where(my_y == 0, 0, t - c0v)
        n_chunks = (n_send + CHUNK - 1) // CHUNK

        out_ref[:, :] = s_ref[:, :]

        barrier = pltpu.get_barrier_semaphore()
        pl.semaphore_signal(
            barrier, inc=1, device_id=partner,
            device_id_type=pl.DeviceIdType.MESH,
        )
        pl.semaphore_wait(barrier, 1)

        descs = []
        for k in range(max_chunks):
            last = k == n_chunks - 1
            so = jnp.maximum(
                jnp.where(last, send_start + n_send - CHUNK,
                          send_start + k * CHUNK), 0)
            do = jnp.maximum(
                jnp.where(last, dst_start + n_send - CHUNK,
                          dst_start + k * CHUNK), 0)
            descs.append(pltpu.make_async_remote_copy(
                src_ref=s_ref.at[pl.ds(so, CHUNK), :],
                dst_ref=out_ref.at[pl.ds(do, CHUNK), :],
                send_sem=send_sems.at[k],
                recv_sem=recv_sems.at[k],
                device_id=partner,
                device_id_type=pl.DeviceIdType.MESH,
            ))

        for k in range(max_chunks):
            @pl.when(k < n_chunks)
            def _(k=k):
                descs[k].start()

        for k in range(max_chunks):
            @pl.when(k < n_chunks)
            def _(k=k):
                descs[k].wait_recv()
                descs[k].wait_send()

    return pl.pallas_call(
        body,
        out_shape=jax.ShapeDtypeStruct((t, d), x.dtype),
        in_specs=[
            pl.BlockSpec(memory_space=pltpu.SMEM),
            pl.BlockSpec(memory_space=pltpu.VMEM),
        ],
        out_specs=pl.BlockSpec(memory_space=pltpu.VMEM),
        scratch_shapes=[
            pltpu.SemaphoreType.DMA((max_chunks,)),
            pltpu.SemaphoreType.DMA((max_chunks,)),
        ],
        compiler_params=pltpu.CompilerParams(collective_id=0),
    )(c0_arr, s)


# baseline (device time: 48562 ns/iter reference)
import jax
import jax.numpy as jnp
from jax import lax
from jax.experimental import pallas as pl
from jax.experimental.pallas import tpu as pltpu

CHUNK = 32


def _exchange(snd, c0_arr):
    t, d = snd.shape
    max_chunks = t // CHUNK

    def body(c0_ref, snd_ref, recv_ref, send_sems, recv_sems):
        my_x = lax.axis_index("x")
        my_y = lax.axis_index("y")
        my_z = lax.axis_index("z")
        partner = (my_x, 1 - my_y, my_z)

        c0v = c0_ref[0]
        n_send = jnp.where(my_y == 0, t - c0v, c0v)
        n_chunks = (n_send + CHUNK - 1) // CHUNK

        barrier = pltpu.get_barrier_semaphore()
        pl.semaphore_signal(
            barrier, inc=1, device_id=partner,
            device_id_type=pl.DeviceIdType.MESH,
        )
        pl.semaphore_wait(barrier, 1)

        descs = []
        for k in range(max_chunks):
            descs.append(pltpu.make_async_remote_copy(
                src_ref=snd_ref.at[pl.ds(k * CHUNK, CHUNK), :],
                dst_ref=recv_ref.at[pl.ds(k * CHUNK, CHUNK), :],
                send_sem=send_sems.at[k],
                recv_sem=recv_sems.at[k],
                device_id=partner,
                device_id_type=pl.DeviceIdType.MESH,
            ))

        for k in range(max_chunks):
            @pl.when(k < n_chunks)
            def _(k=k):
                descs[k].start()

        for k in range(max_chunks):
            @pl.when(k < n_chunks)
            def _(k=k):
                descs[k].wait_recv()
                descs[k].wait_send()

    return pl.pallas_call(
        body,
        out_shape=jax.ShapeDtypeStruct((t, d), snd.dtype),
        in_specs=[
            pl.BlockSpec(memory_space=pltpu.SMEM),
            pl.BlockSpec(memory_space=pltpu.VMEM),
        ],
        out_specs=pl.BlockSpec(memory_space=pltpu.VMEM),
        scratch_shapes=[
            pltpu.SemaphoreType.DMA((max_chunks,)),
            pltpu.SemaphoreType.DMA((max_chunks,)),
        ],
        compiler_params=pltpu.CompilerParams(collective_id=0),
    )(c0_arr, snd)


def kernel(x, dest):
    t, d = x.shape
    my_y = lax.axis_index("y")

    i = jnp.arange(t, dtype=jnp.int32)
    zeros = dest == 0
    c = jnp.cumsum(zeros.astype(jnp.int32))
    c0 = c[-1]
    ones_rank = i - c

    keep = jnp.where(my_y == 0, zeros, ~zeros)
    keep_rank = jnp.where(my_y == 0, c - 1, ones_rank)
    send_rank = jnp.where(my_y == 0, ones_rank, c - 1)
    n_keep = jnp.where(my_y == 0, c0, t - c0)

    send_idx = jnp.zeros((t,), jnp.int32).at[
        jnp.where(keep, t, send_rank)].set(i, mode="drop")
    keep_idx = jnp.zeros((t,), jnp.int32).at[
        jnp.where(keep, keep_rank, t)].set(i, mode="drop")

    snd = jnp.take(x, send_idx, axis=0)
    recv = _exchange(snd, jnp.reshape(c0, (1,)))

    p = i
    n_recv = t - n_keep
    src = jnp.where(
        my_y == 0,
        jnp.where(p < n_keep, keep_idx[jnp.minimum(p, t - 1)], t + p - n_keep),
        jnp.where(p < n_recv, t + p,
                  keep_idx[jnp.clip(p - n_recv, 0, t - 1)]),
    )
    cat = jnp.concatenate([x, recv], axis=0)
    return jnp.take(cat, src, axis=0)
